# baseline (speedup 1.0000x reference)
import functools

import jax
import jax.numpy as jnp
from jax import lax
from jax.experimental import pallas as pl
from jax.experimental.pallas import tpu as pltpu
from jax.experimental.pallas import tpu_sc as plsc

N = 100000
E = 3200000
G = 64
F = 16

NC = 2
NS = 16
NW = NC * NS

CH = 128
K = 5
KH = 10
NCHT = E // CH
NGT = NCHT // K
NGTH = NCHT // KH

RPT = 6256
N_PAD = RPT * NS
R_PAD = N_PAD // 8

BN8 = 736
GRID = R_PAD // BN8

NPCH = N_PAD // CH
GA = 80

_HI = jax.lax.Precision.HIGHEST


def _mesh():
    return plsc.VectorSubcoreMesh(core_axis_name="c", subcore_axis_name="s")


_SC_PARAMS = pltpu.CompilerParams(use_tc_tiling_on_sc=False)


def _sc_hist(ei, aux):
    @functools.partial(
        pl.kernel,
        out_type=jax.ShapeDtypeStruct((NC, N_PAD, F), jnp.float32),
        mesh=_mesh(),
        compiler_params=_SC_PARAMS,
        scratch_types=[
            pltpu.VMEM((KH, CH), jnp.int32),
            pltpu.VMEM((KH, CH), jnp.int32),
            pltpu.VMEM((CH, F), jnp.float32),
            pltpu.VMEM_SHARED((N_PAD, F), jnp.float32),
            pltpu.SemaphoreType.DMA,
            pltpu.SemaphoreType.DMA,
            pltpu.SemaphoreType.DMA,
        ],
    )
    def k(ei_hbm, aux_hbm, out_hbm, cb0, cb1, gbuf, acc, semi0, semi1, sems):
        c = lax.axis_index("c")
        s = lax.axis_index("s")
        wid = c * NS + s
        pltpu.sync_copy(aux_hbm.at[pl.ds(0, RPT)], acc.at[pl.ds(s * RPT, RPT)])
        pltpu.sync_copy(aux_hbm.at[pl.ds(RPT, CH)], gbuf)
        plsc.subcore_barrier()
        glo = wid * NGTH // NW
        ghi = (wid + 1) * NGTH // NW

        @pl.when(glo < ghi)
        def _():
            pltpu.async_copy(ei_hbm.at[1, pl.ds(glo * KH, KH), :], cb0, semi0)

        def process(cb, semi, gnext, cbn, semin):
            @pl.when(gnext < ghi)
            def _():
                pltpu.async_copy(ei_hbm.at[1, pl.ds(gnext * KH, KH), :],
                                 cbn, semin)
            pltpu.make_async_copy(ei_hbm.at[1, pl.ds(0, KH), :],
                                  cb, semi).wait()
            hs = [pltpu.async_copy(gbuf, acc.at[cb.at[b]], sems, add=True)
                  for b in range(KH)]
            for h in hs:
                h.wait()

        @pl.loop(glo, ghi, step=2)
        def _(g):
            process(cb0, semi0, g + 1, cb1, semi1)

            @pl.when(g + 1 < ghi)
            def _():
                process(cb1, semi1, g + 2, cb0, semi0)

        plsc.subcore_barrier()
        pltpu.sync_copy(acc.at[pl.ds(s * RPT, RPT)],
                        out_hbm.at[c, pl.ds(s * RPT, RPT)])

    return k(ei, aux)


def _sc_agg(zfeat, ei, aux):
    @functools.partial(
        pl.kernel,
        out_type=jax.ShapeDtypeStruct((NC, N_PAD, F), jnp.float32),
        mesh=_mesh(),
        compiler_params=_SC_PARAMS,
        scratch_types=[
            pltpu.VMEM((K, CH), jnp.int32),
            pltpu.VMEM((K, CH), jnp.int32),
            pltpu.VMEM((K, CH), jnp.int32),
            pltpu.VMEM((K, CH), jnp.int32),
            pltpu.VMEM((K, CH, F), jnp.float32),
            pltpu.VMEM((K, CH, F), jnp.float32),
            pltpu.VMEM_SHARED((N_PAD, F), jnp.float32),
            pltpu.SemaphoreType.DMA,
            pltpu.SemaphoreType.DMA,
            pltpu.SemaphoreType.DMA,
            pltpu.SemaphoreType.DMA,
        ],
    )
    def k(z_hbm, ei_hbm, aux_hbm, out_hbm,
          cb0, cb1, rb0, rb1, gb0, gb1, acc, semi0, semi1, semg, sems):
        c = lax.axis_index("c")
        s = lax.axis_index("s")
        wid = c * NS + s
        pltpu.sync_copy(aux_hbm.at[pl.ds(0, RPT)], acc.at[pl.ds(s * RPT, RPT)])
        plsc.subcore_barrier()
        glo = wid * NGT // NW
        ghi = (wid + 1) * NGT // NW

        @pl.when(glo < ghi)
        def _():
            pltpu.async_copy(ei_hbm.at[1, pl.ds(glo * K, K), :], cb0, semi0)
            pltpu.async_copy(ei_hbm.at[0, pl.ds(glo * K, K), :], rb0, semi0)

        def process(cb, rb, gb, semi, gnext, cbn, rbn, semin):
            @pl.when(gnext < ghi)
            def _():
                pltpu.async_copy(ei_hbm.at[1, pl.ds(gnext * K, K), :],
                                 cbn, semin)
                pltpu.async_copy(ei_hbm.at[0, pl.ds(gnext * K, K), :],
                                 rbn, semin)
            pltpu.make_async_copy(ei_hbm.at[1, pl.ds(0, K), :],
                                  cb, semi).wait()
            pltpu.make_async_copy(ei_hbm.at[0, pl.ds(0, K), :],
                                  rb, semi).wait()
            hg = [pltpu.async_copy(z_hbm.at[rb.at[b]], gb.at[b], semg)
                  for b in range(K)]
            hs = []
            for b in range(K):
                hg[b].wait()
                hs.append(pltpu.async_copy(gb.at[b], acc.at[cb.at[b]],
                                           sems, add=True))
            for h in hs:
                h.wait()

        @pl.loop(glo, ghi, step=2)
        def _(g):
            process(cb0, rb0, gb0, semi0, g + 1, cb1, rb1, semi1)

            @pl.when(g + 1 < ghi)
            def _():
                process(cb1, rb1, gb1, semi1, g + 2, cb0, rb0, semi0)

        plsc.subcore_barrier()
        pltpu.sync_copy(acc.at[pl.ds(s * RPT, RPT)],
                        out_hbm.at[c, pl.ds(s * RPT, RPT)])

    return k(zfeat, ei, aux)


def _sc_pool(h2feat, batp, aux):
    @functools.partial(
        pl.kernel,
        out_type=jax.ShapeDtypeStruct((NC, 2, GA, F), jnp.float32),
        mesh=_mesh(),
        compiler_params=_SC_PARAMS,
        scratch_types=[
            pltpu.VMEM((K, CH), jnp.int32),
            pltpu.VMEM((K, CH, F), jnp.float32),
            pltpu.VMEM((CH, F), jnp.float32),
            pltpu.VMEM_SHARED((GA, F), jnp.float32),
            pltpu.VMEM_SHARED((GA, F), jnp.float32),
            pltpu.SemaphoreType.DMA,
            pltpu.SemaphoreType.DMA,
            pltpu.SemaphoreType.DMA,
        ],
    )
    def k(h_hbm, bat_hbm, aux_hbm, out_hbm,
          cbuf, gbuf, obuf, accs, accc, semi, semg, sems):
        c = lax.axis_index("c")
        s = lax.axis_index("s")
        wid = c * NS + s
        pltpu.sync_copy(aux_hbm.at[pl.ds(RPT, CH)], obuf)

        @pl.when(s == 0)
        def _():
            pltpu.sync_copy(aux_hbm.at[pl.ds(0, GA)], accs)
            pltpu.sync_copy(aux_hbm.at[pl.ds(0, GA)], accc)

        plsc.subcore_barrier()
        jlo = wid * NPCH // NW
        jhi = (wid + 1) * NPCH // NW

        @pl.loop(jlo, jhi)
        def _(j):
            hc = pltpu.async_copy(bat_hbm.at[pl.ds(j * CH, CH)],
                                  cbuf.at[0], semi)
            hh = pltpu.async_copy(h_hbm.at[pl.ds(j * CH, CH)],
                                  gbuf.at[0], semg)
            hc.wait()
            hh.wait()
            h1 = pltpu.async_copy(gbuf.at[0], accs.at[cbuf.at[0]],
                                  sems, add=True)
            h2 = pltpu.async_copy(obuf, accc.at[cbuf.at[0]],
                                  sems, add=True)
            h1.wait()
            h2.wait()

        plsc.subcore_barrier()

        @pl.when(s == 0)
        def _():
            pltpu.sync_copy(accs, out_hbm.at[c, 0])
            pltpu.sync_copy(accc, out_hbm.at[c, 1])

    return k(h2feat, batp, aux)


def _tc_pre(histl, xlin, w1k):
    def body(h_ref, x_ref, w_ref, dis_ref, z_ref):
        dis = lax.rsqrt(h_ref[0] + h_ref[1] + 1.0)
        dis_ref[...] = dis
        z_ref[...] = jax.lax.dot(x_ref[...], w_ref[...], precision=_HI) * dis

    return pl.pallas_call(
        body,
        grid=(GRID,),
        in_specs=[
            pl.BlockSpec((NC, BN8, 128), lambda i: (0, i, 0)),
            pl.BlockSpec((BN8, 40), lambda i: (i, 0)),
            pl.BlockSpec((40, 128), lambda i: (0, 0)),
        ],
        out_specs=[
            pl.BlockSpec((BN8, 128), lambda i: (i, 0)),
            pl.BlockSpec((BN8, 128), lambda i: (i, 0)),
        ],
        out_shape=[
            jax.ShapeDtypeStruct((R_PAD, 128), jnp.float32),
            jax.ShapeDtypeStruct((R_PAD, 128), jnp.float32),
        ],
    )(histl, xlin, w1k)


def _tc_mid(p, z, dis, w2k, b1l):
    def body(p_ref, z_ref, dis_ref, w_ref, b_ref, z2_ref):
        acc = p_ref[0] + p_ref[1] + z_ref[...]
        h = jnp.maximum(acc * dis_ref[...] + b_ref[...], 0.0)
        z2_ref[...] = jax.lax.dot(h, w_ref[...], precision=_HI) * dis_ref[...]

    return pl.pallas_call(
        body,
        grid=(GRID,),
        in_specs=[
            pl.BlockSpec((NC, BN8, 128), lambda i: (0, i, 0)),
            pl.BlockSpec((BN8, 128), lambda i: (i, 0)),
            pl.BlockSpec((BN8, 128), lambda i: (i, 0)),
            pl.BlockSpec((128, 128), lambda i: (0, 0)),
            pl.BlockSpec((1, 128), lambda i: (0, 0)),
        ],
        out_specs=pl.BlockSpec((BN8, 128), lambda i: (i, 0)),
        out_shape=jax.ShapeDtypeStruct((R_PAD, 128), jnp.float32),
    )(p, z, dis, w2k, b1l)


def _tc_fin(p, z, dis, b2l):
    def body(p_ref, z_ref, dis_ref, b_ref, h_ref):
        acc = p_ref[0] + p_ref[1] + z_ref[...]
        h_ref[...] = jnp.maximum(acc * dis_ref[...] + b_ref[...], 0.0)

    return pl.pallas_call(
        body,
        grid=(GRID,),
        in_specs=[
            pl.BlockSpec((NC, BN8, 128), lambda i: (0, i, 0)),
            pl.BlockSpec((BN8, 128), lambda i: (i, 0)),
            pl.BlockSpec((BN8, 128), lambda i: (i, 0)),
            pl.BlockSpec((1, 128), lambda i: (0, 0)),
        ],
        out_specs=pl.BlockSpec((BN8, 128), lambda i: (i, 0)),
        out_shape=jax.ShapeDtypeStruct((R_PAD, 128), jnp.float32),
    )(p, z, dis, b2l)


def _tc_div(pp):
    def body(p_ref, out_ref):
        s = p_ref[0, 0] + p_ref[1, 0]
        cnt = p_ref[0, 1] + p_ref[1, 1]
        out_ref[...] = (s / jnp.maximum(cnt, 1.0))[:G, :]

    return pl.pallas_call(
        body,
        grid=(1,),
        in_specs=[pl.BlockSpec((NC, 2, GA, F), lambda i: (0, 0, 0, 0))],
        out_specs=pl.BlockSpec((G, F), lambda i: (0, 0)),
        out_shape=jax.ShapeDtypeStruct((G, F), jnp.float32),
    )(pp)


def kernel(x, edge_index, batch, W1, b1, W2, b2):
    batp = jnp.concatenate([batch, jnp.full((N_PAD - N,), G, jnp.int32)])
    aux = jnp.concatenate([jnp.zeros((RPT, F), jnp.float32),
                           jnp.ones((CH, F), jnp.float32)], axis=0)
    eye8 = jnp.eye(8, dtype=jnp.float32)
    w1k = jnp.kron(eye8, W1)
    w2k = jnp.kron(eye8, W2)
    b1l = jnp.tile(b1, 8).reshape(1, 128)
    b2l = jnp.tile(b2, 8).reshape(1, 128)
    xlin = jnp.concatenate(
        [x, jnp.zeros((N_PAD - N, 5), jnp.float32)]).reshape(R_PAD, 40)

    ei3 = edge_index.reshape(2, NCHT, CH)

    hist = _sc_hist(ei3, aux)
    histl = hist.reshape(NC, R_PAD, 128)
    dis, z1 = _tc_pre(histl, xlin, w1k)
    p1 = _sc_agg(z1.reshape(N_PAD, F), ei3, aux)
    z2 = _tc_mid(p1.reshape(NC, R_PAD, 128), z1, dis, w2k, b1l)
    p2 = _sc_agg(z2.reshape(N_PAD, F), ei3, aux)
    h2 = _tc_fin(p2.reshape(NC, R_PAD, 128), z2, dis, b2l)
    pp = _sc_pool(h2.reshape(N_PAD, F), batp, aux)
    return _tc_div(pp)


# speedup vs baseline: 1.0498x; 1.0498x over previous
import functools

import jax
import jax.numpy as jnp
from jax import lax
from jax.experimental import pallas as pl
from jax.experimental.pallas import tpu as pltpu
from jax.experimental.pallas import tpu_sc as plsc

N = 100000
E = 3200000
G = 64
F = 16

NC = 2
NS = 16
NW = NC * NS

CH = 128
K = 10
KH = 20
NCHT = E // CH
NGT = NCHT // K
NGTH = NCHT // KH

RPT = 6256
N_PAD = RPT * NS
R_PAD = N_PAD // 8

BN8 = 736
GRID = R_PAD // BN8

NPCH = N_PAD // CH
GA = 80

_HI = jax.lax.Precision.HIGHEST


def _mesh():
    return plsc.VectorSubcoreMesh(core_axis_name="c", subcore_axis_name="s")


_SC_PARAMS = pltpu.CompilerParams(use_tc_tiling_on_sc=False)


def _sc_hist(ei, aux):
    @functools.partial(
        pl.kernel,
        out_type=jax.ShapeDtypeStruct((NC, N_PAD, F), jnp.float32),
        mesh=_mesh(),
        compiler_params=_SC_PARAMS,
        scratch_types=[
            pltpu.VMEM((KH, CH), jnp.int32),
            pltpu.VMEM((CH, F), jnp.float32),
            pltpu.VMEM_SHARED((N_PAD, F), jnp.float32),
            pltpu.SemaphoreType.DMA,
            pltpu.SemaphoreType.DMA,
        ],
    )
    def k(ei_hbm, aux_hbm, out_hbm, cbuf, gbuf, acc, semi, sems):
        c = lax.axis_index("c")
        s = lax.axis_index("s")
        wid = c * NS + s
        pltpu.sync_copy(aux_hbm.at[pl.ds(0, RPT)], acc.at[pl.ds(s * RPT, RPT)])
        pltpu.sync_copy(aux_hbm.at[pl.ds(RPT, CH)], gbuf)
        plsc.subcore_barrier()
        glo = wid * NGTH // NW
        ghi = (wid + 1) * NGTH // NW

        @pl.loop(glo, ghi)
        def _(g):
            c0 = g * KH
            hc = [pltpu.async_copy(ei_hbm.at[1, c0 + b], cbuf.at[b], semi)
                  for b in range(KH)]
            hs = []
            for b in range(KH):
                hc[b].wait()
                hs.append(pltpu.async_copy(gbuf, acc.at[cbuf.at[b]], sems,
                                           add=True))
            for h in hs:
                h.wait()

        plsc.subcore_barrier()
        pltpu.sync_copy(acc.at[pl.ds(s * RPT, RPT)],
                        out_hbm.at[c, pl.ds(s * RPT, RPT)])

    return k(ei, aux)


def _sc_agg(zfeat, ei, aux):
    @functools.partial(
        pl.kernel,
        out_type=jax.ShapeDtypeStruct((NC, N_PAD, F), jnp.float32),
        mesh=_mesh(),
        compiler_params=_SC_PARAMS,
        scratch_types=[
            pltpu.VMEM((K, CH), jnp.int32),
            pltpu.VMEM((K, CH), jnp.int32),
            pltpu.VMEM((K, CH, F), jnp.float32),
            pltpu.VMEM_SHARED((N_PAD, F), jnp.float32),
            pltpu.SemaphoreType.DMA,
            pltpu.SemaphoreType.DMA,
            pltpu.SemaphoreType.DMA,
        ],
    )
    def k(z_hbm, ei_hbm, aux_hbm, out_hbm,
          cbuf, rbuf, gbuf, acc, semi, semg, sems):
        c = lax.axis_index("c")
        s = lax.axis_index("s")
        wid = c * NS + s
        pltpu.sync_copy(aux_hbm.at[pl.ds(0, RPT)], acc.at[pl.ds(s * RPT, RPT)])
        plsc.subcore_barrier()
        glo = wid * NGT // NW
        ghi = (wid + 1) * NGT // NW

        @pl.loop(glo, ghi)
        def _(g):
            c0 = g * K
            hc = []
            hr = []
            for b in range(K):
                hc.append(pltpu.async_copy(ei_hbm.at[1, c0 + b],
                                           cbuf.at[b], semi))
                hr.append(pltpu.async_copy(ei_hbm.at[0, c0 + b],
                                           rbuf.at[b], semi))
            hg = []
            for b in range(K):
                hc[b].wait()
                hr[b].wait()
                hg.append(pltpu.async_copy(z_hbm.at[rbuf.at[b]],
                                           gbuf.at[b], semg))
            hs = []
            for b in range(K):
                hg[b].wait()
                hs.append(pltpu.async_copy(gbuf.at[b], acc.at[cbuf.at[b]],
                                           sems, add=True))
            for h in hs:
                h.wait()

        plsc.subcore_barrier()
        pltpu.sync_copy(acc.at[pl.ds(s * RPT, RPT)],
                        out_hbm.at[c, pl.ds(s * RPT, RPT)])

    return k(zfeat, ei, aux)


def _sc_pool(h2feat, batp, aux):
    @functools.partial(
        pl.kernel,
        out_type=jax.ShapeDtypeStruct((NC, 2, GA, F), jnp.float32),
        mesh=_mesh(),
        compiler_params=_SC_PARAMS,
        scratch_types=[
            pltpu.VMEM((K, CH), jnp.int32),
            pltpu.VMEM((K, CH, F), jnp.float32),
            pltpu.VMEM((CH, F), jnp.float32),
            pltpu.VMEM_SHARED((GA, F), jnp.float32),
            pltpu.VMEM_SHARED((GA, F), jnp.float32),
            pltpu.SemaphoreType.DMA,
            pltpu.SemaphoreType.DMA,
            pltpu.SemaphoreType.DMA,
        ],
    )
    def k(h_hbm, bat_hbm, aux_hbm, out_hbm,
          cbuf, gbuf, obuf, accs, accc, semi, semg, sems):
        c = lax.axis_index("c")
        s = lax.axis_index("s")
        wid = c * NS + s
        pltpu.sync_copy(aux_hbm.at[pl.ds(RPT, CH)], obuf)

        @pl.when(s == 0)
        def _():
            pltpu.sync_copy(aux_hbm.at[pl.ds(0, GA)], accs)
            pltpu.sync_copy(aux_hbm.at[pl.ds(0, GA)], accc)

        plsc.subcore_barrier()
        jlo = wid * NPCH // NW
        jhi = (wid + 1) * NPCH // NW

        @pl.loop(jlo, jhi)
        def _(j):
            hc = pltpu.async_copy(bat_hbm.at[pl.ds(j * CH, CH)],
                                  cbuf.at[0], semi)
            hh = pltpu.async_copy(h_hbm.at[pl.ds(j * CH, CH)],
                                  gbuf.at[0], semg)
            hc.wait()
            hh.wait()
            h1 = pltpu.async_copy(gbuf.at[0], accs.at[cbuf.at[0]],
                                  sems, add=True)
            h2 = pltpu.async_copy(obuf, accc.at[cbuf.at[0]],
                                  sems, add=True)
            h1.wait()
            h2.wait()

        plsc.subcore_barrier()

        @pl.when(s == 0)
        def _():
            pltpu.sync_copy(accs, out_hbm.at[c, 0])
            pltpu.sync_copy(accc, out_hbm.at[c, 1])

    return k(h2feat, batp, aux)


def _tc_pre(histl, xlin, w1k):
    def body(h_ref, x_ref, w_ref, dis_ref, z_ref):
        dis = lax.rsqrt(h_ref[0] + h_ref[1] + 1.0)
        dis_ref[...] = dis
        z_ref[...] = jax.lax.dot(x_ref[...], w_ref[...], precision=_HI) * dis

    return pl.pallas_call(
        body,
        grid=(GRID,),
        in_specs=[
            pl.BlockSpec((NC, BN8, 128), lambda i: (0, i, 0)),
            pl.BlockSpec((BN8, 40), lambda i: (i, 0)),
            pl.BlockSpec((40, 128), lambda i: (0, 0)),
        ],
        out_specs=[
            pl.BlockSpec((BN8, 128), lambda i: (i, 0)),
            pl.BlockSpec((BN8, 128), lambda i: (i, 0)),
        ],
        out_shape=[
            jax.ShapeDtypeStruct((R_PAD, 128), jnp.float32),
            jax.ShapeDtypeStruct((R_PAD, 128), jnp.float32),
        ],
    )(histl, xlin, w1k)


def _tc_mid(p, z, dis, w2k, b1l):
    def body(p_ref, z_ref, dis_ref, w_ref, b_ref, z2_ref):
        acc = p_ref[0] + p_ref[1] + z_ref[...]
        h = jnp.maximum(acc * dis_ref[...] + b_ref[...], 0.0)
        z2_ref[...] = jax.lax.dot(h, w_ref[...], precision=_HI) * dis_ref[...]

    return pl.pallas_call(
        body,
        grid=(GRID,),
        in_specs=[
            pl.BlockSpec((NC, BN8, 128), lambda i: (0, i, 0)),
            pl.BlockSpec((BN8, 128), lambda i: (i, 0)),
            pl.BlockSpec((BN8, 128), lambda i: (i, 0)),
            pl.BlockSpec((128, 128), lambda i: (0, 0)),
            pl.BlockSpec((1, 128), lambda i: (0, 0)),
        ],
        out_specs=pl.BlockSpec((BN8, 128), lambda i: (i, 0)),
        out_shape=jax.ShapeDtypeStruct((R_PAD, 128), jnp.float32),
    )(p, z, dis, w2k, b1l)


def _tc_fin(p, z, dis, b2l):
    def body(p_ref, z_ref, dis_ref, b_ref, h_ref):
        acc = p_ref[0] + p_ref[1] + z_ref[...]
        h_ref[...] = jnp.maximum(acc * dis_ref[...] + b_ref[...], 0.0)

    return pl.pallas_call(
        body,
        grid=(GRID,),
        in_specs=[
            pl.BlockSpec((NC, BN8, 128), lambda i: (0, i, 0)),
            pl.BlockSpec((BN8, 128), lambda i: (i, 0)),
            pl.BlockSpec((BN8, 128), lambda i: (i, 0)),
            pl.BlockSpec((1, 128), lambda i: (0, 0)),
        ],
        out_specs=pl.BlockSpec((BN8, 128), lambda i: (i, 0)),
        out_shape=jax.ShapeDtypeStruct((R_PAD, 128), jnp.float32),
    )(p, z, dis, b2l)


def _tc_div(pp):
    def body(p_ref, out_ref):
        s = p_ref[0, 0] + p_ref[1, 0]
        cnt = p_ref[0, 1] + p_ref[1, 1]
        out_ref[...] = (s / jnp.maximum(cnt, 1.0))[:G, :]

    return pl.pallas_call(
        body,
        grid=(1,),
        in_specs=[pl.BlockSpec((NC, 2, GA, F), lambda i: (0, 0, 0, 0))],
        out_specs=pl.BlockSpec((G, F), lambda i: (0, 0)),
        out_shape=jax.ShapeDtypeStruct((G, F), jnp.float32),
    )(pp)


def kernel(x, edge_index, batch, W1, b1, W2, b2):
    batp = jnp.concatenate([batch, jnp.full((N_PAD - N,), G, jnp.int32)])
    aux = jnp.concatenate([jnp.zeros((RPT, F), jnp.float32),
                           jnp.ones((CH, F), jnp.float32)], axis=0)
    eye8 = jnp.eye(8, dtype=jnp.float32)
    w1k = jnp.kron(eye8, W1)
    w2k = jnp.kron(eye8, W2)
    b1l = jnp.tile(b1, 8).reshape(1, 128)
    b2l = jnp.tile(b2, 8).reshape(1, 128)
    xlin = jnp.concatenate(
        [x, jnp.zeros((N_PAD - N, 5), jnp.float32)]).reshape(R_PAD, 40)

    ei3 = edge_index.reshape(2, NCHT, CH)

    hist = _sc_hist(ei3, aux)
    histl = hist.reshape(NC, R_PAD, 128)
    dis, z1 = _tc_pre(histl, xlin, w1k)
    p1 = _sc_agg(z1.reshape(N_PAD, F), ei3, aux)
    z2 = _tc_mid(p1.reshape(NC, R_PAD, 128), z1, dis, w2k, b1l)
    p2 = _sc_agg(z2.reshape(N_PAD, F), ei3, aux)
    h2 = _tc_fin(p2.reshape(NC, R_PAD, 128), z2, dis, b2l)
    pp = _sc_pool(h2.reshape(N_PAD, F), batp, aux)
    return _tc_div(pp)
